# baseline (speedup 1.0000x reference)
import functools

import jax
import jax.numpy as jnp
from jax import lax
from jax.experimental import pallas as pl
from jax.experimental.pallas import tpu as pltpu
from jax.experimental.pallas import tpu_sc as plsc

N = 10000
H = 32
K = 15
KP = 16
CIN = 128
COUT = 128
KP_EXTENT = 1.2
E = N * H

NC = 2
NS = 16
NW = NC * NS

CH = 128
NCHUNK = E // CH
MAXC = (NCHUNK + NW - 1) // NW

B = 200
BH = B * H
GB = N // B
G = 8
GH = G * H
GKP = G * KP
NG = B // G


def _sc_gather_body(table_hbm, inds_hbm, xn_hbm, idx_v, rows_v, sem):
    wid = lax.axis_index("s") * NC + lax.axis_index("c")

    def body(i, carry):
        c = wid + i * NW

        @pl.when(c < NCHUNK)
        def _():
            off = pl.multiple_of(c * CH, CH)
            pltpu.sync_copy(inds_hbm.at[pl.ds(off, CH)], idx_v)
            pltpu.async_copy(table_hbm.at[idx_v], rows_v, sem).wait()
            pltpu.sync_copy(rows_v, xn_hbm.at[pl.ds(off, CH)])

        return carry

    lax.fori_loop(0, MAXC, body, 0)


def _sc_gather(table, inds):
    mesh = plsc.VectorSubcoreMesh(core_axis_name="c", subcore_axis_name="s")
    fn = pl.kernel(
        _sc_gather_body,
        mesh=mesh,
        out_type=jax.ShapeDtypeStruct((E, CIN), jnp.float32),
        scratch_types=[
            pltpu.VMEM((CH,), jnp.int32),
            pltpu.VMEM((CH, CIN), jnp.float32),
            pltpu.SemaphoreType.DMA,
        ],
    )
    return fn(table, inds)


def _tc_body(mgt_ref, w_ref, xn_ref, out_ref):
    raw = xn_ref[...]
    wi = lax.bitcast_convert_type(raw[:, 0:64], jnp.int32)
    f_even = lax.bitcast_convert_type(wi << 16, jnp.float32)
    f_odd = lax.bitcast_convert_type(wi & jnp.int32(-65536), jnp.float32)
    feats = jnp.concatenate([f_even, f_odd], axis=1)
    c3 = raw[:, 64:67]
    hi3 = c3.astype(jnp.bfloat16).astype(jnp.float32)
    lo3 = c3 - hi3
    saug = jnp.concatenate([hi3, lo3, raw[:, 67:69]], axis=1)
    mgt = mgt_ref[...]
    c_i = lax.broadcasted_iota(jnp.int32, (GH, GKP), 0)
    r_i = lax.broadcasted_iota(jnp.int32, (GH, GKP), 1)
    maskf = ((c_i // H) == (r_i // KP)).astype(jnp.float32)
    parts = []
    for g in range(NG):
        sg = saug[g * GH:(g + 1) * GH, :]
        mg = mgt[:, g * 3 * GKP:(g + 1) * 3 * GKP]
        d = jnp.dot(sg, mg, preferred_element_type=jnp.float32)
        dx = d[:, 0:GKP]
        dy = d[:, GKP:2 * GKP]
        dz = d[:, 2 * GKP:3 * GKP]
        sqb = dx * dx + dy * dy + dz * dz
        wb = jnp.maximum(
            1.0 - jnp.sqrt(sqb) * (1.0 / KP_EXTENT),
            0.0) * maskf
        fg = feats[g * GH:(g + 1) * GH, :]
        parts.append(lax.dot_general(
            wb, fg, (((0,), (0,)), ((), ())),
            preferred_element_type=jnp.float32))
    a2 = jnp.concatenate(parts, axis=0)
    a = a2.reshape(B, KP * CIN)
    out_ref[...] = jnp.dot(a, w_ref[...], preferred_element_type=jnp.float32)


def _tc_call(mgt, wflat, xn):
    return pl.pallas_call(
        _tc_body,
        grid=(GB,),
        in_specs=[
            pl.BlockSpec((8, B * KP * 3), lambda i: (0, i)),
            pl.BlockSpec((KP * CIN, COUT), lambda i: (0, 0)),
            pl.BlockSpec((BH, CIN), lambda i: (i, 0)),
        ],
        out_specs=pl.BlockSpec((B, COUT), lambda i: (i, 0)),
        out_shape=jax.ShapeDtypeStruct((N, COUT), jnp.float32),
    )(mgt, wflat, xn)


def _pack_table(x, s_pts):
    xb = x.astype(jnp.bfloat16)
    packedf = lax.bitcast_convert_type(
        xb.reshape(N, 64, 2), jnp.float32)
    ones = jnp.ones((N, 2), jnp.float32)
    row = jnp.concatenate(
        [packedf, s_pts, ones, jnp.zeros((N, 59), jnp.float32)], axis=1)
    shadow = jnp.zeros((1, 128), jnp.float32)
    shadow = shadow.at[0, 64:67].set(1e6).at[0, 67:69].set(1.0)
    return jnp.concatenate([row, shadow], axis=0)


def _make_maugt(q_pts, K_points):
    t = q_pts[:, None, :] + K_points[None, :, :]
    tpad = jnp.concatenate(
        [t, jnp.full((N, 1, 3), -1e15, jnp.float32)], axis=1)
    ngrp = N * KP // GKP
    trt = tpad.reshape(ngrp, GKP, 3).transpose(0, 2, 1)
    rows_eye = jnp.broadcast_to(
        jnp.eye(3, dtype=jnp.float32)[:, None, :, None],
        (3, ngrp, 3, GKP))
    thi = (-trt).astype(jnp.bfloat16).astype(jnp.float32)
    tlo = (-trt) - thi
    return jnp.concatenate(
        [rows_eye, rows_eye, thi[None], tlo[None]], axis=0).reshape(8, -1)


def kernel(q_pts, s_pts, neighb_inds, x, K_points, W):
    table = _pack_table(x, s_pts)
    inds = neighb_inds.astype(jnp.int32).reshape(E)
    xn = _sc_gather(table, inds)
    mgt = _make_maugt(q_pts, K_points)
    perm = jnp.arange(CIN).reshape(64, 2).T.reshape(CIN)
    wperm = W[:, perm, :]
    wflat = jnp.concatenate(
        [wperm, jnp.zeros((1, CIN, COUT), jnp.float32)],
        axis=0).reshape(KP * CIN, COUT)
    return _tc_call(mgt, wflat, xn)


# speedup vs baseline: 1.1699x; 1.1699x over previous
import functools

import jax
import jax.numpy as jnp
from jax import lax
from jax.experimental import pallas as pl
from jax.experimental.pallas import tpu as pltpu
from jax.experimental.pallas import tpu_sc as plsc

N = 10000
H = 32
K = 15
KP = 16
CIN = 128
COUT = 128
KP_EXTENT = 1.2
E = N * H

NC = 2
NS = 16
NW = NC * NS

NSPLIT = 2
ECH = E // NSPLIT
NCHP = N // NSPLIT

CH = 128
NCHUNK = ECH // CH
MAXC = (NCHUNK + NW - 1) // NW

B = 200
BH = B * H
GB = N // B
G = 8
GH = G * H
GKP = G * KP
NG = B // G


def _sc_gather_body(table_hbm, inds_hbm, xn_hbm, idx_v, rows_v, sem):
    wid = lax.axis_index("s") * NC + lax.axis_index("c")

    def body(i, carry):
        c = wid + i * NW

        @pl.when(c < NCHUNK)
        def _():
            off = pl.multiple_of(c * CH, CH)
            pltpu.sync_copy(inds_hbm.at[pl.ds(off, CH)], idx_v)
            pltpu.async_copy(table_hbm.at[idx_v], rows_v, sem).wait()
            pltpu.sync_copy(rows_v, xn_hbm.at[pl.ds(off, CH)])

        return carry

    lax.fori_loop(0, MAXC, body, 0)


def _sc_gather(table, inds):
    mesh = plsc.VectorSubcoreMesh(core_axis_name="c", subcore_axis_name="s")
    fn = pl.kernel(
        _sc_gather_body,
        mesh=mesh,
        out_type=jax.ShapeDtypeStruct((ECH, CIN), jnp.float32),
        scratch_types=[
            pltpu.VMEM((CH,), jnp.int32),
            pltpu.VMEM((CH, CIN), jnp.float32),
            pltpu.SemaphoreType.DMA,
        ],
    )
    return fn(table, inds)


def _tc_body(mgt_ref, w_ref, xn_ref, out_ref):
    raw = xn_ref[...]
    wi = lax.bitcast_convert_type(raw[:, 0:64], jnp.int32)
    f_even = lax.bitcast_convert_type(wi << 16, jnp.float32)
    f_odd = lax.bitcast_convert_type(wi & jnp.int32(-65536), jnp.float32)
    feats = jnp.concatenate([f_even, f_odd], axis=1)
    c3 = raw[:, 64:67]
    hi3 = c3.astype(jnp.bfloat16).astype(jnp.float32)
    lo3 = c3 - hi3
    saug = jnp.concatenate([hi3, lo3, raw[:, 67:69]], axis=1)
    mgt = mgt_ref[...]
    c_i = lax.broadcasted_iota(jnp.int32, (GH, GKP), 0)
    r_i = lax.broadcasted_iota(jnp.int32, (GH, GKP), 1)
    maskf = ((c_i // H) == (r_i // KP)).astype(jnp.float32)
    parts = []
    for g in range(NG):
        sg = saug[g * GH:(g + 1) * GH, :]
        mg = mgt[:, g * 3 * GKP:(g + 1) * 3 * GKP]
        d = jnp.dot(sg, mg, preferred_element_type=jnp.float32)
        dx = d[:, 0:GKP]
        dy = d[:, GKP:2 * GKP]
        dz = d[:, 2 * GKP:3 * GKP]
        sqb = dx * dx + dy * dy + dz * dz
        wb = jnp.maximum(
            1.0 - jnp.sqrt(sqb) * (1.0 / KP_EXTENT),
            0.0) * maskf
        fg = feats[g * GH:(g + 1) * GH, :]
        parts.append(lax.dot_general(
            wb, fg, (((0,), (0,)), ((), ())),
            preferred_element_type=jnp.float32))
    a2 = jnp.concatenate(parts, axis=0)
    a = a2.reshape(B, KP * CIN)
    out_ref[...] = jnp.dot(a, w_ref[...], preferred_element_type=jnp.float32)


def _tc_call(mgt, wflat, xn, ci):
    blk0 = ci * (NCHP // B)
    return pl.pallas_call(
        _tc_body,
        grid=(NCHP // B,),
        in_specs=[
            pl.BlockSpec((8, B * KP * 3), lambda i: (0, blk0 + i)),
            pl.BlockSpec((KP * CIN, COUT), lambda i: (0, 0)),
            pl.BlockSpec((BH, CIN), lambda i: (i, 0)),
        ],
        out_specs=pl.BlockSpec((B, COUT), lambda i: (i, 0)),
        out_shape=jax.ShapeDtypeStruct((NCHP, COUT), jnp.float32),
    )(mgt, wflat, xn)


def _pack_table(x, s_pts):
    xb = x.astype(jnp.bfloat16)
    packedf = lax.bitcast_convert_type(
        xb.reshape(N, 64, 2), jnp.float32)
    ones = jnp.ones((N, 2), jnp.float32)
    row = jnp.concatenate(
        [packedf, s_pts, ones, jnp.zeros((N, 59), jnp.float32)], axis=1)
    shadow = jnp.zeros((1, 128), jnp.float32)
    shadow = shadow.at[0, 64:67].set(1e6).at[0, 67:69].set(1.0)
    return jnp.concatenate([row, shadow], axis=0)


def _make_maugt(q_pts, K_points):
    t = q_pts[:, None, :] + K_points[None, :, :]
    tpad = jnp.concatenate(
        [t, jnp.full((N, 1, 3), -1e15, jnp.float32)], axis=1)
    ngrp = N * KP // GKP
    trt = tpad.reshape(ngrp, GKP, 3).transpose(0, 2, 1)
    rows_eye = jnp.broadcast_to(
        jnp.eye(3, dtype=jnp.float32)[:, None, :, None],
        (3, ngrp, 3, GKP))
    thi = (-trt).astype(jnp.bfloat16).astype(jnp.float32)
    tlo = (-trt) - thi
    return jnp.concatenate(
        [rows_eye, rows_eye, thi[None], tlo[None]], axis=0).reshape(8, -1)


def kernel(q_pts, s_pts, neighb_inds, x, K_points, W):
    table = _pack_table(x, s_pts)
    inds = neighb_inds.astype(jnp.int32).reshape(E)
    mgt = _make_maugt(q_pts, K_points)
    perm = jnp.arange(CIN).reshape(64, 2).T.reshape(CIN)
    wperm = W[:, perm, :]
    wflat = jnp.concatenate(
        [wperm, jnp.zeros((1, CIN, COUT), jnp.float32)],
        axis=0).reshape(KP * CIN, COUT)
    xns = [_sc_gather(table, lax.slice_in_dim(inds, ci * ECH, (ci + 1) * ECH))
           for ci in range(NSPLIT)]
    outs = [_tc_call(mgt, wflat, xns[ci], ci) for ci in range(NSPLIT)]
    return jnp.concatenate(outs, axis=0)


# speedup vs baseline: 1.2298x; 1.0512x over previous
import functools

import jax
import jax.numpy as jnp
from jax import lax
from jax.experimental import pallas as pl
from jax.experimental.pallas import tpu as pltpu
from jax.experimental.pallas import tpu_sc as plsc

N = 10000
H = 32
K = 15
KP = 16
CIN = 128
COUT = 128
KP_EXTENT = 1.2
E = N * H

NC = 2
NS = 16
NW = NC * NS

NSPLIT = 4
ECH = E // NSPLIT
NCHP = N // NSPLIT

CH = 128
NCHUNK = ECH // CH
MAXC = (NCHUNK + NW - 1) // NW

B = 200
BH = B * H
GB = N // B
G = 8
GH = G * H
GKP = G * KP
NG = B // G


def _sc_gather_body(table_hbm, inds_hbm, xn_hbm, idx_v, rows_v, sem):
    wid = lax.axis_index("s") * NC + lax.axis_index("c")

    def body(i, carry):
        c = wid + i * NW

        @pl.when(c < NCHUNK)
        def _():
            off = pl.multiple_of(c * CH, CH)
            pltpu.sync_copy(inds_hbm.at[pl.ds(off, CH)], idx_v)
            pltpu.async_copy(table_hbm.at[idx_v], rows_v, sem).wait()
            pltpu.sync_copy(rows_v, xn_hbm.at[pl.ds(off, CH)])

        return carry

    lax.fori_loop(0, MAXC, body, 0)


def _sc_gather(table, inds):
    mesh = plsc.VectorSubcoreMesh(core_axis_name="c", subcore_axis_name="s")
    fn = pl.kernel(
        _sc_gather_body,
        mesh=mesh,
        out_type=jax.ShapeDtypeStruct((ECH, CIN), jnp.float32),
        scratch_types=[
            pltpu.VMEM((CH,), jnp.int32),
            pltpu.VMEM((CH, CIN), jnp.float32),
            pltpu.SemaphoreType.DMA,
        ],
    )
    return fn(table, inds)


def _tc_body(mgt_ref, w_ref, xn_ref, out_ref):
    raw = xn_ref[...]
    wi = lax.bitcast_convert_type(raw[:, 0:64], jnp.int32)
    f_even = lax.bitcast_convert_type(wi << 16, jnp.float32)
    f_odd = lax.bitcast_convert_type(wi & jnp.int32(-65536), jnp.float32)
    feats = jnp.concatenate([f_even, f_odd], axis=1)
    c3 = raw[:, 64:67]
    hi3 = c3.astype(jnp.bfloat16).astype(jnp.float32)
    lo3 = c3 - hi3
    saug = jnp.concatenate([hi3, lo3, raw[:, 67:69]], axis=1)
    mgt = mgt_ref[...]
    c_i = lax.broadcasted_iota(jnp.int32, (GH, GKP), 0)
    r_i = lax.broadcasted_iota(jnp.int32, (GH, GKP), 1)
    maskf = ((c_i // H) == (r_i // KP)).astype(jnp.float32)
    parts = []
    for g in range(NG):
        sg = saug[g * GH:(g + 1) * GH, :]
        mg = mgt[:, g * 3 * GKP:(g + 1) * 3 * GKP]
        d = jnp.dot(sg, mg, preferred_element_type=jnp.float32)
        dx = d[:, 0:GKP]
        dy = d[:, GKP:2 * GKP]
        dz = d[:, 2 * GKP:3 * GKP]
        sqb = dx * dx + dy * dy + dz * dz
        wb = jnp.maximum(
            1.0 - jnp.sqrt(sqb) * (1.0 / KP_EXTENT),
            0.0) * maskf
        fg = feats[g * GH:(g + 1) * GH, :]
        parts.append(lax.dot_general(
            wb, fg, (((0,), (0,)), ((), ())),
            preferred_element_type=jnp.float32))
    a2 = jnp.concatenate(parts, axis=0)
    a = a2.reshape(B, KP * CIN)
    out_ref[...] = jnp.dot(a, w_ref[...], preferred_element_type=jnp.float32)


def _tc_call(mgt, wflat, xn, ci):
    blk0 = ci * (NCHP // B)
    return pl.pallas_call(
        _tc_body,
        grid=(NCHP // B,),
        in_specs=[
            pl.BlockSpec((8, B * KP * 3), lambda i: (0, blk0 + i)),
            pl.BlockSpec((KP * CIN, COUT), lambda i: (0, 0)),
            pl.BlockSpec((BH, CIN), lambda i: (i, 0)),
        ],
        out_specs=pl.BlockSpec((B, COUT), lambda i: (i, 0)),
        out_shape=jax.ShapeDtypeStruct((NCHP, COUT), jnp.float32),
    )(mgt, wflat, xn)


def _pack_table(x, s_pts):
    xb = x.astype(jnp.bfloat16)
    packedf = lax.bitcast_convert_type(
        xb.reshape(N, 64, 2), jnp.float32)
    ones = jnp.ones((N, 2), jnp.float32)
    row = jnp.concatenate(
        [packedf, s_pts, ones, jnp.zeros((N, 59), jnp.float32)], axis=1)
    shadow = jnp.zeros((1, 128), jnp.float32)
    shadow = shadow.at[0, 64:67].set(1e6).at[0, 67:69].set(1.0)
    return jnp.concatenate([row, shadow], axis=0)


def _make_maugt(q_pts, K_points):
    t = q_pts[:, None, :] + K_points[None, :, :]
    tpad = jnp.concatenate(
        [t, jnp.full((N, 1, 3), -1e15, jnp.float32)], axis=1)
    ngrp = N * KP // GKP
    trt = tpad.reshape(ngrp, GKP, 3).transpose(0, 2, 1)
    rows_eye = jnp.broadcast_to(
        jnp.eye(3, dtype=jnp.float32)[:, None, :, None],
        (3, ngrp, 3, GKP))
    thi = (-trt).astype(jnp.bfloat16).astype(jnp.float32)
    tlo = (-trt) - thi
    return jnp.concatenate(
        [rows_eye, rows_eye, thi[None], tlo[None]], axis=0).reshape(8, -1)


def kernel(q_pts, s_pts, neighb_inds, x, K_points, W):
    table = _pack_table(x, s_pts)
    inds = neighb_inds.astype(jnp.int32).reshape(E)
    mgt = _make_maugt(q_pts, K_points)
    perm = jnp.arange(CIN).reshape(64, 2).T.reshape(CIN)
    wperm = W[:, perm, :]
    wflat = jnp.concatenate(
        [wperm, jnp.zeros((1, CIN, COUT), jnp.float32)],
        axis=0).reshape(KP * CIN, COUT)
    xns = [_sc_gather(table, lax.slice_in_dim(inds, ci * ECH, (ci + 1) * ECH))
           for ci in range(NSPLIT)]
    outs = [_tc_call(mgt, wflat, xns[ci], ci) for ci in range(NSPLIT)]
    return jnp.concatenate(outs, axis=0)
